# baseline (speedup 1.0000x reference)
import functools

import jax
import jax.numpy as jnp
from jax import lax
from jax.experimental import pallas as pl
from jax.experimental.pallas import tpu as pltpu
from jax.experimental.pallas import tpu_sc as plsc

BATCH = 16384
EMBED_DIM = 64
CLS_DIM = 32
NUM_ACT = 1000000
NUM_CLS = 1000
NC = 2
NS = 16
NW = NC * NS
B_PER_W = BATCH // NW
CHUNK = 128
N_CHUNK = B_PER_W // CHUNK
LANES = 16


def _fill_idx(ids_v, idx_all, n_rows, table_stride):
    def body(d, _):
        off = d * table_stride
        def inner(k, _):
            v = ids_v[pl.ds(k * LANES, LANES)]
            idx_all[d, pl.ds(k * LANES, LANES)] = v + off
            return 0
        return lax.fori_loop(0, B_PER_W // LANES, inner, 0)
    lax.fori_loop(0, n_rows, body, 0)


@functools.partial(
    pl.kernel,
    out_type=(
        jax.ShapeDtypeStruct((EMBED_DIM, BATCH), jnp.float32),
        jax.ShapeDtypeStruct((CLS_DIM, BATCH), jnp.float32),
    ),
    mesh=plsc.VectorSubcoreMesh(core_axis_name="c", subcore_axis_name="s"),
    scratch_types=[
        pltpu.VMEM((B_PER_W,), jnp.int32),
        pltpu.VMEM((B_PER_W,), jnp.int32),
        pltpu.VMEM((EMBED_DIM, B_PER_W), jnp.int32),
        pltpu.VMEM((CLS_DIM, B_PER_W), jnp.int32),
        pltpu.VMEM((EMBED_DIM, B_PER_W), jnp.float32),
        pltpu.VMEM((CLS_DIM, B_PER_W), jnp.float32),
        pltpu.SemaphoreType.DMA,
        pltpu.SemaphoreType.DMA,
    ],
)
def _sc_gather(ids_hbm, cls_hbm, embT_hbm, clsT_hbm, actT_out, clsT_out,
               ids_v, clsids_v, aidx, cidx, actT_v, clsT_v, sem_a, sem_c):
    wid = lax.axis_index("s") * NC + lax.axis_index("c")
    base = wid * B_PER_W
    pltpu.sync_copy(ids_hbm.at[pl.ds(base, B_PER_W)], ids_v)
    pltpu.sync_copy(cls_hbm.at[pl.ds(base, B_PER_W)], clsids_v)

    _fill_idx(ids_v, aidx, EMBED_DIM, NUM_ACT)
    _fill_idx(clsids_v, cidx, CLS_DIM, NUM_CLS)

    def make_fire(tab, idx_all, dst, sem):
        def fire(d, carry):
            for j in range(N_CHUNK):
                pltpu.async_copy(
                    tab.at[idx_all.at[d, pl.ds(j * CHUNK, CHUNK)]],
                    dst.at[d, pl.ds(j * CHUNK, CHUNK)], sem)
            return carry
        return fire
    lax.fori_loop(0, EMBED_DIM, make_fire(embT_hbm, aidx, actT_v, sem_a), 0)
    lax.fori_loop(0, CLS_DIM, make_fire(clsT_hbm, cidx, clsT_v, sem_c), 0)
    pltpu.make_async_copy(actT_out, actT_v, sem_a).wait()
    pltpu.make_async_copy(clsT_out, clsT_v, sem_c).wait()
    pltpu.sync_copy(actT_v, actT_out.at[:, pl.ds(base, B_PER_W)])
    pltpu.sync_copy(clsT_v, clsT_out.at[:, pl.ds(base, B_PER_W)])


def _mm_body(actT_ref, clsT_ref, w1t_ref, w2t_ref, bt_ref, o_ref):
    acc = jnp.dot(w1t_ref[...], actT_ref[...],
                  preferred_element_type=jnp.float32,
                  precision=lax.Precision.HIGHEST)
    acc += jnp.dot(w2t_ref[...], clsT_ref[...],
                   preferred_element_type=jnp.float32,
                   precision=lax.Precision.HIGHEST)
    o_ref[...] = acc + bt_ref[...]


def _tc_project(actT, clsT, w1t, w2t, bt):
    blk = 2048
    grid = (BATCH // blk,)
    return pl.pallas_call(
        _mm_body,
        grid=grid,
        in_specs=[
            pl.BlockSpec((EMBED_DIM, blk), lambda i: (0, i)),
            pl.BlockSpec((CLS_DIM, blk), lambda i: (0, i)),
            pl.BlockSpec((EMBED_DIM, EMBED_DIM), lambda i: (0, 0)),
            pl.BlockSpec((EMBED_DIM, CLS_DIM), lambda i: (0, 0)),
            pl.BlockSpec((EMBED_DIM, 1), lambda i: (0, 0)),
        ],
        out_specs=pl.BlockSpec((EMBED_DIM, blk), lambda i: (0, i)),
        out_shape=jax.ShapeDtypeStruct((EMBED_DIM, BATCH), jnp.float32),
    )(actT, clsT, w1t, w2t, bt)


def kernel(activity_ids, activity_classes, embedding, class_embedding, W, b):
    ids = activity_ids.astype(jnp.int32)
    cls = activity_classes.astype(jnp.int32)
    emb_flat = embedding.T.reshape(NUM_ACT * EMBED_DIM)
    cls_flat = class_embedding.T.reshape(NUM_CLS * CLS_DIM)
    actT, clsT = _sc_gather(ids, cls, emb_flat, cls_flat)
    wt = W.T
    outT = _tc_project(actT, clsT, wt[:, :EMBED_DIM], wt[:, EMBED_DIM:],
                       b.reshape(EMBED_DIM, 1))
    return outT.T


# speedup vs baseline: 9.5442x; 9.5442x over previous
import functools

import jax
import jax.numpy as jnp
from jax import lax
from jax.experimental import pallas as pl
from jax.experimental.pallas import tpu as pltpu
from jax.experimental.pallas import tpu_sc as plsc

BATCH = 16384
EMBED_DIM = 64
CLS_DIM = 32
NUM_ACT = 1000000
NC = 2
NS = 16
NW = NC * NS
B_PER_W = BATCH // NW
CHUNK = 128
N_CHUNK = B_PER_W // CHUNK

TL = 2048
TGRID = (NUM_ACT + TL - 1) // TL


def _tr_body(in_ref, o_ref):
    o_ref[:, :EMBED_DIM] = in_ref[...].T


def _tc_pairize(embT):
    return pl.pallas_call(
        _tr_body,
        grid=(TGRID,),
        in_specs=[pl.BlockSpec((EMBED_DIM, TL), lambda i: (0, i))],
        out_specs=pl.BlockSpec((TL, 128), lambda i: (i, 0)),
        out_shape=jax.ShapeDtypeStruct((NUM_ACT, 128), jnp.float32),
    )(embT)


@functools.partial(
    pl.kernel,
    out_type=(
        jax.ShapeDtypeStruct((BATCH, 128), jnp.float32),
        jax.ShapeDtypeStruct((BATCH, 128), jnp.float32),
    ),
    mesh=plsc.VectorSubcoreMesh(core_axis_name="c", subcore_axis_name="s"),
    compiler_params=pltpu.CompilerParams(use_tc_tiling_on_sc=True),
    scratch_types=[
        pltpu.VMEM((B_PER_W,), jnp.int32),
        pltpu.VMEM((B_PER_W,), jnp.int32),
        pltpu.VMEM((B_PER_W, 128), jnp.float32),
        pltpu.VMEM((B_PER_W // 2, 128), jnp.float32),
        pltpu.SemaphoreType.DMA,
        pltpu.SemaphoreType.DMA,
    ],
)
def _sc_gather(ids_hbm, cls_hbm, emb_hbm, clsemb_hbm, act_out, cls_out,
               ids_v, clsids_v, act_rows, cls_rows, sem_a, sem_c):
    wid = lax.axis_index("s") * NC + lax.axis_index("c")
    base = wid * B_PER_W
    pltpu.sync_copy(ids_hbm.at[pl.ds(base, B_PER_W)], ids_v)
    pltpu.sync_copy(cls_hbm.at[pl.ds(base, B_PER_W)], clsids_v)
    act_copies = []
    for j in range(N_CHUNK):
        act_copies.append(pltpu.async_copy(
            emb_hbm.at[ids_v.at[pl.ds(j * CHUNK, CHUNK)]],
            act_rows.at[pl.ds(j * CHUNK, CHUNK)], sem_a))
    for r in range(2):
        cls_copies = []
        for j in range(2):
            cls_copies.append(pltpu.async_copy(
                clsemb_hbm.at[clsids_v.at[pl.ds((2 * r + j) * CHUNK, CHUNK)]],
                cls_rows.at[pl.ds(j * CHUNK, CHUNK)], sem_c))
        for c in cls_copies:
            c.wait()
        pltpu.sync_copy(cls_rows,
                        cls_out.at[pl.ds(base + r * (B_PER_W // 2),
                                         B_PER_W // 2)])
    for c in act_copies:
        c.wait()
    pltpu.sync_copy(act_rows, act_out.at[pl.ds(base, B_PER_W)])


def _mm_body(act2_ref, cls4_ref, clsm_ref, w1_ref, w2_ref, b_ref,
             o_ref):
    clsm = clsm_ref[...]
    act = act2_ref[:, :EMBED_DIM]
    c = cls4_ref[...]
    cls_sel = c[:, 0:CLS_DIM] * (clsm == 0.0)
    cls_sel += c[:, CLS_DIM:2 * CLS_DIM] * (clsm == 1.0)
    cls_sel += c[:, 2 * CLS_DIM:3 * CLS_DIM] * (clsm == 2.0)
    cls_sel += c[:, 3 * CLS_DIM:] * (clsm == 3.0)
    acc = jnp.dot(act, w1_ref[...],
                  preferred_element_type=jnp.float32,
                  precision=lax.Precision.HIGHEST)
    acc += jnp.dot(cls_sel, w2_ref[...],
                   preferred_element_type=jnp.float32,
                   precision=lax.Precision.HIGHEST)
    o_ref[...] = acc + b_ref[...]


def _tc_project(act2, cls4, clsm, w1, w2, b2d):
    blk = 2048
    grid = (BATCH // blk,)
    return pl.pallas_call(
        _mm_body,
        grid=grid,
        in_specs=[
            pl.BlockSpec((blk, 128), lambda i: (i, 0)),
            pl.BlockSpec((blk, 128), lambda i: (i, 0)),
            pl.BlockSpec((blk, 1), lambda i: (i, 0)),
            pl.BlockSpec((EMBED_DIM, EMBED_DIM), lambda i: (0, 0)),
            pl.BlockSpec((CLS_DIM, EMBED_DIM), lambda i: (0, 0)),
            pl.BlockSpec((1, EMBED_DIM), lambda i: (0, 0)),
        ],
        out_specs=pl.BlockSpec((blk, EMBED_DIM), lambda i: (i, 0)),
        out_shape=jax.ShapeDtypeStruct((BATCH, EMBED_DIM), jnp.float32),
    )(act2, cls4, clsm, w1, w2, b2d)


def kernel(activity_ids, activity_classes, embedding, class_embedding, W, b):
    ids = activity_ids.astype(jnp.int32)
    cls = activity_classes.astype(jnp.int32)
    emb2 = _tc_pairize(embedding.T)
    cls2 = class_embedding.reshape(250, 128)
    act2, cls4 = _sc_gather(ids, cls // 4, emb2, cls2)
    clsm = (cls % 4).astype(jnp.float32).reshape(BATCH, 1)
    return _tc_project(act2, cls4, clsm,
                       W[:EMBED_DIM], W[EMBED_DIM:], b.reshape(1, EMBED_DIM))
